# Initial kernel scaffold; baseline (speedup 1.0000x reference)
import jax
import jax.numpy as jnp
from jax.experimental import pallas as pl


def kernel(w0, b0, w1, b1, w2, b2, w3, b3, w4, b4, w5, b5, w6, b6, w7, b7, w8, b8, img):
    raise NotImplementedError("write your pallas kernel here")


# speedup vs baseline: 1.1301x; 1.1301x over previous
import functools

import numpy as np

import jax
import jax.numpy as jnp
from jax.experimental import pallas as pl
from jax.experimental.pallas import tpu as pltpu

_LAYER_CHANNELS = ((3, 64), (64, 64), (64, 64), (64, 32), (32, 32),
                   (32, 32), (32, 16), (16, 16), (16, 16))
_NUM_LAYERS = len(_LAYER_CHANNELS)
_OUT_C = 16
_LANE = 128
_HALF = 64
_NTAPS = 9
_KDIM = _NTAPS * _LANE


def _round_up(x, m):
    return ((x + m - 1) // m) * m


def _downsample_half(x_nhwc):
    n, h, w, c = x_nhwc.shape
    ho, wo = h // 2, w // 2
    x = x_nhwc[:, :2 * ho, :2 * wo, :].reshape(n, ho, 2, wo, 2, c)
    return x.mean(axis=(2, 4))


def _plan(scale_dims):
    pre = 24
    blocks = []
    cur = pre
    for (h, w) in scale_dims:
        wp = _round_up(w + 1, 2)
        gg = wp // 2
        img_g = h * wp // 2
        rows = _round_up(gg + img_g + gg, 8)
        blocks.append(dict(h=h, w=w, wp=wp, block_start=cur,
                           img_start=cur + gg, img_g=img_g, rows=rows))
        cur += rows
    m2 = cur + 8
    return blocks, m2


def _build_mask(blocks, m2):
    mask = np.zeros((m2, _LANE), np.float32)
    for b in blocks:
        h, w, wp = b["h"], b["w"], b["wp"]
        p = np.arange(h * wp)
        valid = (p % wp < w).astype(np.float32)
        g0 = b["img_start"]
        mask[g0:g0 + b["img_g"], :_HALF] = valid[0::2, None]
        mask[g0:g0 + b["img_g"], _HALF:] = valid[1::2, None]
    return mask


def _build_wblk(w, cin, cout):
    z = jnp.zeros((_NTAPS, 2, _HALF, 2, _HALF), jnp.float32)
    for dy in (-1, 0, 1):
        for dx in (-1, 0, 1):
            for o in (0, 1):
                k = (o + dx) // 2
                slot = (o + dx) % 2
                t = 3 * (dy + 1) + (k + 1)
                z = z.at[t, slot, :cin, o, :cout].set(w[dy + 1, dx + 1])
    return z.reshape(_KDIM, _LANE)


def _pyramid_kernel(chunks, img_ref, mask_ref, w_ref, b_ref, out_ref,
                    act_ref, slab_ref):
    m2 = img_ref.shape[0]
    act_ref[...] = img_ref[...]
    first = chunks[0][0]
    last = chunks[-1][0] + chunks[-1][1]
    slab_ref[0:first, :] = jnp.zeros((first, _KDIM), jnp.bfloat16)
    slab_ref[last:m2, :] = jnp.zeros((m2 - last, _KDIM), jnp.bfloat16)

    def assemble():
        for (g0, rows, wp) in chunks:
            for t, (dy, k) in enumerate(((dy, k) for dy in (-1, 0, 1)
                                         for k in (-1, 0, 1))):
                s = dy * (wp // 2) + k
                slab_ref[g0:g0 + rows, t * _LANE:(t + 1) * _LANE] = (
                    act_ref[g0 + s:g0 + s + rows, :])

    def layer_math(l):
        y = jnp.dot(slab_ref[...], w_ref[l],
                    preferred_element_type=jnp.float32)
        y = y + b_ref[l]
        y = jnp.maximum(y, 0.1 * y)
        return y * mask_ref[...]

    def body(l, carry):
        assemble()
        act_ref[...] = layer_math(l).astype(jnp.bfloat16)
        return carry

    jax.lax.fori_loop(0, _NUM_LAYERS - 1, body, 0)
    assemble()
    out_ref[...] = layer_math(_NUM_LAYERS - 1)


@functools.partial(jax.jit, static_argnames=())
def _forward(params, img_nchw):
    img = jnp.transpose(img_nchw, (0, 2, 3, 1)).astype(jnp.float32)
    n = img.shape[0]

    pyr = [img]
    for _ in range(4):
        pyr.append(_downsample_half(pyr[-1]))
    scale_dims = tuple((int(p.shape[1]), int(p.shape[2])) for p in pyr)
    blocks, m2 = _plan(scale_dims)

    pieces = [jnp.zeros((n, blocks[0]["block_start"], _LANE), jnp.bfloat16)]
    for p, b in zip(pyr, blocks):
        h, w, wp = b["h"], b["w"], b["wp"]
        q = jnp.pad(p.astype(jnp.bfloat16), ((0, 0), (0, 0), (0, wp - w), (0, 0)))
        q = q.reshape(n, h * wp // 2, 2, 3)
        q = jnp.pad(q, ((0, 0), (0, 0), (0, 0), (0, _HALF - 3)))
        q = q.reshape(n, h * wp // 2, _LANE)
        gg = b["img_start"] - b["block_start"]
        tail = b["rows"] - gg - b["img_g"]
        pieces.append(jnp.zeros((n, gg, _LANE), jnp.bfloat16))
        pieces.append(q)
        pieces.append(jnp.zeros((n, tail, _LANE), jnp.bfloat16))
    pieces.append(jnp.zeros((n, m2 - blocks[-1]["block_start"] - blocks[-1]["rows"],
                             _LANE), jnp.bfloat16))
    img_packed = jnp.concatenate(pieces, axis=1)

    mask = jnp.asarray(_build_mask(blocks, m2))

    w_stack = jnp.stack([
        _build_wblk(w, cin, cout)
        for (w, _), (cin, cout) in zip(params, _LAYER_CHANNELS)
    ]).astype(jnp.bfloat16)
    b_stack = jnp.stack([
        jnp.pad(b, (0, _HALF - b.shape[0]))
        for (_, b) in params])
    b_stack = jnp.concatenate([b_stack, b_stack], axis=-1).reshape(
        _NUM_LAYERS, 1, _LANE)

    chunks = tuple((b["block_start"], b["rows"], b["wp"]) for b in blocks)
    kfn = functools.partial(_pyramid_kernel, chunks)

    out = pl.pallas_call(
        kfn,
        grid=(n,),
        in_specs=[
            pl.BlockSpec((None, m2, _LANE), lambda i: (i, 0, 0)),
            pl.BlockSpec((m2, _LANE), lambda i: (0, 0)),
            pl.BlockSpec((_NUM_LAYERS, _KDIM, _LANE), lambda i: (0, 0, 0)),
            pl.BlockSpec((_NUM_LAYERS, 1, _LANE), lambda i: (0, 0, 0)),
        ],
        out_specs=pl.BlockSpec((None, m2, _LANE), lambda i: (i, 0, 0)),
        out_shape=jax.ShapeDtypeStruct((n, m2, _LANE), jnp.float32),
        scratch_shapes=[
            pltpu.VMEM((m2, _LANE), jnp.bfloat16),
            pltpu.VMEM((m2, _KDIM), jnp.bfloat16),
        ],
        compiler_params=pltpu.CompilerParams(
            dimension_semantics=("parallel",),
            vmem_limit_bytes=48 * 1024 * 1024),
    )(img_packed, mask, w_stack, b_stack)

    feats = []
    for b in blocks:
        h, w, wp = b["h"], b["w"], b["wp"]
        f = out[:, b["img_start"]:b["img_start"] + b["img_g"], :]
        f = f.reshape(n, h * wp // 2, 2, _HALF).reshape(n, h, wp, _HALF)
        f = f[:, :, :w, :_OUT_C]
        feats.append(jnp.transpose(f, (0, 3, 1, 2)))
    return feats


def kernel(w0, b0, w1, b1, w2, b2, w3, b3, w4, b4,
           w5, b5, w6, b6, w7, b7, w8, b8, img):
    params = [(w0, b0), (w1, b1), (w2, b2), (w3, b3), (w4, b4),
              (w5, b5), (w6, b6), (w7, b7), (w8, b8)]
    return _forward(params, img)


# speedup vs baseline: 1.5312x; 1.3550x over previous
import functools

import numpy as np

import jax
import jax.numpy as jnp
from jax.experimental import pallas as pl
from jax.experimental.pallas import tpu as pltpu

_LAYER_CHANNELS = ((3, 64), (64, 64), (64, 64), (64, 32), (32, 32),
                   (32, 32), (32, 16), (16, 16), (16, 16))
_NUM_LAYERS = len(_LAYER_CHANNELS)
_OUT_C = 16
_LANE = 128
_HALF = 64
_NTAPS = 9
_KDIM = _NTAPS * _LANE


def _round_up(x, m):
    return ((x + m - 1) // m) * m


def _downsample_half(x_nhwc):
    n, h, w, c = x_nhwc.shape
    ho, wo = h // 2, w // 2
    x = x_nhwc[:, :2 * ho, :2 * wo, :].reshape(n, ho, 2, wo, 2, c)
    return x.mean(axis=(2, 4))


def _plan(scale_dims):
    pre = 32
    blocks = []
    cur = pre
    for (h, w) in scale_dims:
        wp = _round_up(w + 1, 16)
        gg = wp // 2
        img_g = h * wp // 2
        rows = _round_up(gg + img_g + gg, 8)
        blocks.append(dict(h=h, w=w, wp=wp, block_start=cur,
                           img_start=cur + gg, img_g=img_g, rows=rows))
        cur += rows
    m2 = cur + 16
    return blocks, m2


def _build_mask(blocks, m2):
    mask = np.zeros((m2, _LANE), np.float32)
    for b in blocks:
        h, w, wp = b["h"], b["w"], b["wp"]
        p = np.arange(h * wp)
        valid = (p % wp < w).astype(np.float32)
        g0 = b["img_start"]
        mask[g0:g0 + b["img_g"], :_HALF] = valid[0::2, None]
        mask[g0:g0 + b["img_g"], _HALF:] = valid[1::2, None]
    return mask


def _build_wblk(w, cin, cout):
    z = jnp.zeros((_NTAPS, 2, _HALF, 2, _HALF), jnp.float32)
    for dy in (-1, 0, 1):
        for dx in (-1, 0, 1):
            for o in (0, 1):
                k = (o + dx) // 2
                slot = (o + dx) % 2
                t = 3 * (dy + 1) + (k + 1)
                z = z.at[t, slot, :cin, o, :cout].set(w[dy + 1, dx + 1])
    return z.reshape(_KDIM, _LANE)


def _pyramid_kernel(chunks, img_ref, mask_ref, w_ref, b_ref, out_ref,
                    act_ref, actm_ref, actp_ref, slab_ref):
    m2 = img_ref.shape[0]
    act_ref[...] = img_ref[...].astype(jnp.float32)
    first = chunks[0][0]
    last = chunks[-1][0] + chunks[-1][1]
    slab_ref[0:first, :] = jnp.zeros((first, _KDIM), jnp.float32)
    slab_ref[last:m2, :] = jnp.zeros((m2 - last, _KDIM), jnp.float32)
    actm_ref[0:8, :] = jnp.zeros((8, _LANE), jnp.float32)
    actp_ref[m2 - 8:m2, :] = jnp.zeros((8, _LANE), jnp.float32)

    def assemble():
        actp_ref[0:m2 - 8, :] = act_ref[1:m2 - 7, :]
        actm_ref[8:m2, :] = act_ref[7:m2 - 1, :]
        for (g0, rows, wp) in chunks:
            for t, (dy, k) in enumerate(((dy, k) for dy in (-1, 0, 1)
                                         for k in (-1, 0, 1))):
                s = dy * (wp // 2)
                src = actm_ref if k == -1 else (actp_ref if k == 1 else act_ref)
                slab_ref[g0:g0 + rows, t * _LANE:(t + 1) * _LANE] = (
                    src[g0 + s:g0 + s + rows, :])

    def layer_math(l):
        y = jnp.dot(slab_ref[...], w_ref[l],
                    preferred_element_type=jnp.float32)
        y = y + b_ref[l]
        y = jnp.maximum(y, 0.1 * y)
        return y * mask_ref[...]

    def body(l, carry):
        assemble()
        act_ref[...] = layer_math(l)
        return carry

    jax.lax.fori_loop(0, _NUM_LAYERS - 1, body, 0)
    assemble()
    out_ref[...] = layer_math(_NUM_LAYERS - 1).astype(jnp.bfloat16)


@functools.partial(jax.jit, static_argnames=())
def _forward(params, img_nchw):
    img = jnp.transpose(img_nchw, (0, 2, 3, 1)).astype(jnp.float32)
    n = img.shape[0]

    pyr = [img]
    for _ in range(4):
        pyr.append(_downsample_half(pyr[-1]))
    scale_dims = tuple((int(p.shape[1]), int(p.shape[2])) for p in pyr)
    blocks, m2 = _plan(scale_dims)

    pieces = [jnp.zeros((n, blocks[0]["block_start"], _LANE), jnp.bfloat16)]
    for p, b in zip(pyr, blocks):
        h, w, wp = b["h"], b["w"], b["wp"]
        q = jnp.pad(p.astype(jnp.bfloat16), ((0, 0), (0, 0), (0, wp - w), (0, 0)))
        q = q.reshape(n, h * wp // 2, 2, 3)
        q = jnp.pad(q, ((0, 0), (0, 0), (0, 0), (0, _HALF - 3)))
        q = q.reshape(n, h * wp // 2, _LANE)
        gg = b["img_start"] - b["block_start"]
        tail = b["rows"] - gg - b["img_g"]
        pieces.append(jnp.zeros((n, gg, _LANE), jnp.bfloat16))
        pieces.append(q)
        pieces.append(jnp.zeros((n, tail, _LANE), jnp.bfloat16))
    pieces.append(jnp.zeros((n, m2 - blocks[-1]["block_start"] - blocks[-1]["rows"],
                             _LANE), jnp.bfloat16))
    img_packed = jnp.concatenate(pieces, axis=1)

    mask = jnp.asarray(_build_mask(blocks, m2))

    w_stack = jnp.stack([
        _build_wblk(w, cin, cout)
        for (w, _), (cin, cout) in zip(params, _LAYER_CHANNELS)
    ])
    b_stack = jnp.stack([
        jnp.pad(b, (0, _HALF - b.shape[0]))
        for (_, b) in params])
    b_stack = jnp.concatenate([b_stack, b_stack], axis=-1).reshape(
        _NUM_LAYERS, 1, _LANE)

    chunks = tuple((b["block_start"], b["rows"], b["wp"]) for b in blocks)
    kfn = functools.partial(_pyramid_kernel, chunks)

    out = pl.pallas_call(
        kfn,
        grid=(n,),
        in_specs=[
            pl.BlockSpec((None, m2, _LANE), lambda i: (i, 0, 0)),
            pl.BlockSpec((m2, _LANE), lambda i: (0, 0)),
            pl.BlockSpec((_NUM_LAYERS, _KDIM, _LANE), lambda i: (0, 0, 0)),
            pl.BlockSpec((_NUM_LAYERS, 1, _LANE), lambda i: (0, 0, 0)),
        ],
        out_specs=pl.BlockSpec((None, m2, _LANE), lambda i: (i, 0, 0)),
        out_shape=jax.ShapeDtypeStruct((n, m2, _LANE), jnp.bfloat16),
        scratch_shapes=[
            pltpu.VMEM((m2, _LANE), jnp.float32),
            pltpu.VMEM((m2, _LANE), jnp.float32),
            pltpu.VMEM((m2, _LANE), jnp.float32),
            pltpu.VMEM((m2, _KDIM), jnp.float32),
        ],
        compiler_params=pltpu.CompilerParams(
            dimension_semantics=("parallel",),
            vmem_limit_bytes=48 * 1024 * 1024),
    )(img_packed, mask, w_stack, b_stack)

    feats = []
    for b in blocks:
        h, w, wp = b["h"], b["w"], b["wp"]
        f = out[:, b["img_start"]:b["img_start"] + b["img_g"], :]
        f = f.reshape(n, h * wp // 2, 2, _HALF).reshape(n, h, wp, _HALF)
        f = f[:, :, :w, :_OUT_C].astype(jnp.float32)
        feats.append(jnp.transpose(f, (0, 3, 1, 2)))
    return feats


def kernel(w0, b0, w1, b1, w2, b2, w3, b3, w4, b4,
           w5, b5, w6, b6, w7, b7, w8, b8, img):
    params = [(w0, b0), (w1, b1), (w2, b2), (w3, b3), (w4, b4),
              (w5, b5), (w6, b6), (w7, b7), (w8, b8)]
    return _forward(params, img)
